# baseline (speedup 1.0000x reference)
import functools

import jax
import jax.numpy as jnp
from jax import lax
from jax.experimental import pallas as pl
from jax.experimental.pallas import tpu as pltpu
from jax.experimental.pallas import tpu_sc as plsc

NUM_ROWS = 1000000
EMBED_DIM = 64
BATCH = 16384

L = 16
DMA_CHUNK = 16


def _make_sc_kernel():
    info = plsc.get_sparse_core_info()
    nc, ns = info.num_cores, info.num_subcores
    nw = nc * ns
    bpw = BATCH // nw
    nblk = bpw // L

    mesh = plsc.VectorSubcoreMesh(core_axis_name="c", subcore_axis_name="s")

    @functools.partial(
        pl.kernel,
        mesh=mesh,
        out_type=jax.ShapeDtypeStruct((BATCH,), jnp.float32),
        scratch_types=[
            pltpu.VMEM((bpw,), jnp.int32),
            pltpu.VMEM((bpw,), jnp.int32),
            pltpu.VMEM((EMBED_DIM, bpw), jnp.float32),
            pltpu.VMEM((EMBED_DIM, bpw), jnp.float32),
            pltpu.VMEM((bpw,), jnp.float32),
            pltpu.VMEM((bpw,), jnp.float32),
            pltpu.VMEM((L,), jnp.float32),
            pltpu.VMEM((bpw,), jnp.float32),
            pltpu.SemaphoreType.DMA,
            pltpu.SemaphoreType.DMA,
            pltpu.SemaphoreType.DMA,
        ],
        compiler_params=pltpu.CompilerParams(
            needs_layout_passes=False, use_tc_tiling_on_sc=False),
    )
    def sc_kernel(uid_hbm, eid_hbm, utt_hbm, ett_hbm, ub_hbm, eb_hbm, gb_hbm,
                  out_hbm, uid_v, eid_v, u_cols, e_cols, ub_v, eb_v, gb_v,
                  scores_v, sem0, sem1, semb):
        wid = lax.axis_index("s") * nc + lax.axis_index("c")
        base = wid * bpw

        pltpu.sync_copy(uid_hbm.at[pl.ds(base, bpw)], uid_v)
        pltpu.sync_copy(eid_hbm.at[pl.ds(base, bpw)], eid_v)
        cub = pltpu.async_copy(ub_hbm.at[uid_v], ub_v, semb)
        ceb = pltpu.async_copy(eb_hbm.at[eid_v], eb_v, semb)
        pltpu.sync_copy(gb_hbm.at[pl.ds(0, 1)], gb_v.at[pl.ds(0, 1)])

        for c0 in range(0, EMBED_DIM, DMA_CHUNK):
            cps = []
            for d in range(c0, c0 + DMA_CHUNK):
                cps.append(pltpu.async_copy(
                    utt_hbm.at[d].at[uid_v], u_cols.at[d], sem0))
                cps.append(pltpu.async_copy(
                    ett_hbm.at[d].at[eid_v], e_cols.at[d], sem1))
            for cp in cps:
                cp.wait()
        cub.wait()
        ceb.wait()

        gb = gb_v[pl.ds(0, L)][0]

        def block(j, _):
            sl = pl.ds(j * L, L)
            acc = jnp.full((L,), gb, jnp.float32)
            for d in range(EMBED_DIM):
                acc = acc + u_cols[d, sl] * e_cols[d, sl]
            acc = acc + ub_v[sl] + eb_v[sl]
            scores_v[sl] = acc
            return _

        lax.fori_loop(0, nblk, block, None)
        pltpu.sync_copy(scores_v, out_hbm.at[pl.ds(base, bpw)])

    return sc_kernel


_sc_kernel = _make_sc_kernel()


def kernel(user_ids, event_ids, user_table, event_table, user_bias,
           event_bias, global_bias):
    uid = user_ids.astype(jnp.int32)
    eid = event_ids.astype(jnp.int32)
    return _sc_kernel(uid, eid, user_table.T, event_table.T,
                      user_bias.T.reshape(-1), event_bias.T.reshape(-1),
                      global_bias)


# speedup vs baseline: 9.6220x; 9.6220x over previous
import functools

import jax
import jax.numpy as jnp
from jax import lax
from jax.experimental import pallas as pl
from jax.experimental.pallas import tpu as pltpu
from jax.experimental.pallas import tpu_sc as plsc

NUM_ROWS = 1000000
EMBED_DIM = 64
BATCH = 16384
PADDED = 128

L = 16
TCHUNK = 2048


def _tp_body(x_ref, o_ref):
    o_ref[:, 0:EMBED_DIM] = jnp.transpose(x_ref[...], (1, 0))


_tc_transpose = pl.pallas_call(
    _tp_body,
    grid=((NUM_ROWS + TCHUNK - 1) // TCHUNK,),
    in_specs=[pl.BlockSpec((EMBED_DIM, TCHUNK), lambda i: (0, i))],
    out_specs=pl.BlockSpec((TCHUNK, PADDED), lambda i: (i, 0)),
    out_shape=jax.ShapeDtypeStruct((NUM_ROWS, PADDED), jnp.float32),
)


def _make_sc_kernel():
    info = plsc.get_sparse_core_info()
    nc, ns = info.num_cores, info.num_subcores
    nw = nc * ns
    bpw = BATCH // nw
    half = bpw // 2
    nblk = half // L

    mesh = plsc.VectorSubcoreMesh(core_axis_name="c", subcore_axis_name="s")

    @functools.partial(
        pl.kernel,
        mesh=mesh,
        out_type=jax.ShapeDtypeStruct((BATCH,), jnp.float32),
        scratch_types=[
            pltpu.VMEM((bpw,), jnp.int32),
            pltpu.VMEM((bpw,), jnp.int32),
            pltpu.VMEM((half, PADDED), jnp.float32),
            pltpu.VMEM((half, PADDED), jnp.float32),
            pltpu.VMEM((bpw,), jnp.float32),
            pltpu.VMEM((bpw,), jnp.float32),
            pltpu.VMEM((L,), jnp.float32),
            pltpu.VMEM((bpw,), jnp.float32),
            pltpu.SemaphoreType.DMA,
            pltpu.SemaphoreType.DMA,
            pltpu.SemaphoreType.DMA,
        ],
        compiler_params=pltpu.CompilerParams(
            needs_layout_passes=False, use_tc_tiling_on_sc=False),
    )
    def sc_kernel(uid_hbm, eid_hbm, ut_hbm, et_hbm, ub_hbm, eb_hbm, gb_hbm,
                  out_hbm, uid_v, eid_v, u_rows, e_rows, ub_v, eb_v, gb_v,
                  scores_v, sem0, sem1, semb):
        wid = lax.axis_index("s") * nc + lax.axis_index("c")
        base = wid * bpw

        pltpu.sync_copy(uid_hbm.at[pl.ds(base, bpw)], uid_v)
        pltpu.sync_copy(eid_hbm.at[pl.ds(base, bpw)], eid_v)
        cub = pltpu.async_copy(ub_hbm.at[0].at[uid_v], ub_v, semb)
        ceb = pltpu.async_copy(eb_hbm.at[0].at[eid_v], eb_v, semb)
        pltpu.sync_copy(gb_hbm.at[pl.ds(0, 1)], gb_v.at[pl.ds(0, 1)])

        gb = gb_v[pl.ds(0, L)][0]
        lane = lax.iota(jnp.int32, L)

        for h in range(2):
            cu = pltpu.async_copy(
                ut_hbm.at[uid_v.at[pl.ds(h * half, half)]], u_rows, sem0)
            ce = pltpu.async_copy(
                et_hbm.at[eid_v.at[pl.ds(h * half, half)]], e_rows, sem1)
            cu.wait()
            ce.wait()

            def block(j, _):
                row = jnp.full((L,), j * L, jnp.int32) + lane
                acc = jnp.full((L,), gb, jnp.float32)
                for d in range(EMBED_DIM):
                    col = jnp.full((L,), d, jnp.int32)
                    gu = plsc.load_gather(u_rows, [row, col])
                    ge = plsc.load_gather(e_rows, [row, col])
                    acc = acc + gu * ge
                scores_v[pl.ds(h * half + j * L, L)] = acc
                return _

            lax.fori_loop(0, nblk, block, None)

        def bias_block(j, _):
            sl = pl.ds(j * L, L)
            scores_v[sl] = scores_v[sl] + ub_v[sl] + eb_v[sl]
            return _

        cub.wait()
        ceb.wait()
        lax.fori_loop(0, bpw // L, bias_block, None)
        pltpu.sync_copy(scores_v, out_hbm.at[pl.ds(base, bpw)])

    return sc_kernel


_sc_kernel = _make_sc_kernel()


def kernel(user_ids, event_ids, user_table, event_table, user_bias,
           event_bias, global_bias):
    uid = user_ids.astype(jnp.int32)
    eid = event_ids.astype(jnp.int32)
    ut2 = _tc_transpose(user_table.T)
    et2 = _tc_transpose(event_table.T)
    return _sc_kernel(uid, eid, ut2, et2,
                      user_bias.T, event_bias.T, global_bias)


# speedup vs baseline: 12.3160x; 1.2800x over previous
import functools

import jax
import jax.numpy as jnp
from jax import lax
from jax.experimental import pallas as pl
from jax.experimental.pallas import tpu as pltpu
from jax.experimental.pallas import tpu_sc as plsc

NUM_ROWS = 1000000
EMBED_DIM = 64
BATCH = 16384
PADDED = 128

L = 16
TCHUNK = 4096


def _tp_body(x_ref, o_ref):
    eye = jnp.eye(EMBED_DIM, dtype=jnp.float32)
    o_ref[:, 0:EMBED_DIM] = jax.lax.dot_general(
        x_ref[...], eye, (((0,), (0,)), ((), ())),
        preferred_element_type=jnp.float32)


_tc_transpose = pl.pallas_call(
    _tp_body,
    grid=((NUM_ROWS + TCHUNK - 1) // TCHUNK,),
    in_specs=[pl.BlockSpec((EMBED_DIM, TCHUNK), lambda i: (0, i))],
    out_specs=pl.BlockSpec((TCHUNK, PADDED), lambda i: (i, 0)),
    out_shape=jax.ShapeDtypeStruct((NUM_ROWS, PADDED), jnp.float32),
)


def _make_sc_kernel():
    info = plsc.get_sparse_core_info()
    nc, ns = info.num_cores, info.num_subcores
    nw = nc * ns
    bpw = BATCH // nw
    half = bpw // 2
    nblk = half // L

    mesh = plsc.VectorSubcoreMesh(core_axis_name="c", subcore_axis_name="s")

    @functools.partial(
        pl.kernel,
        mesh=mesh,
        out_type=jax.ShapeDtypeStruct((BATCH,), jnp.float32),
        scratch_types=[
            pltpu.VMEM((bpw,), jnp.int32),
            pltpu.VMEM((bpw,), jnp.int32),
            pltpu.VMEM((half, PADDED), jnp.float32),
            pltpu.VMEM((half, PADDED), jnp.float32),
            pltpu.VMEM((bpw,), jnp.float32),
            pltpu.VMEM((bpw,), jnp.float32),
            pltpu.VMEM((L,), jnp.float32),
            pltpu.VMEM((bpw,), jnp.float32),
            pltpu.SemaphoreType.DMA,
            pltpu.SemaphoreType.DMA,
            pltpu.SemaphoreType.DMA,
        ],
        compiler_params=pltpu.CompilerParams(
            needs_layout_passes=False, use_tc_tiling_on_sc=False),
    )
    def sc_kernel(uid_hbm, eid_hbm, ut_hbm, et_hbm, ub_hbm, eb_hbm, gb_hbm,
                  out_hbm, uid_v, eid_v, u_rows, e_rows, ub_v, eb_v, gb_v,
                  scores_v, sem0, sem1, semb):
        wid = lax.axis_index("s") * nc + lax.axis_index("c")
        base = wid * bpw

        pltpu.sync_copy(uid_hbm.at[pl.ds(base, bpw)], uid_v)
        pltpu.sync_copy(eid_hbm.at[pl.ds(base, bpw)], eid_v)
        cub = pltpu.async_copy(ub_hbm.at[0].at[uid_v], ub_v, semb)
        ceb = pltpu.async_copy(eb_hbm.at[0].at[eid_v], eb_v, semb)
        pltpu.sync_copy(gb_hbm.at[pl.ds(0, 1)], gb_v.at[pl.ds(0, 1)])

        gb = gb_v[pl.ds(0, L)][0]
        lane = lax.iota(jnp.int32, L)

        for h in range(2):
            cu = pltpu.async_copy(
                ut_hbm.at[uid_v.at[pl.ds(h * half, half)]], u_rows, sem0)
            ce = pltpu.async_copy(
                et_hbm.at[eid_v.at[pl.ds(h * half, half)]], e_rows, sem1)
            cu.wait()
            ce.wait()

            def block(j, _):
                row = jnp.full((L,), j * L, jnp.int32) + lane
                acc = jnp.full((L,), gb, jnp.float32)
                for d in range(EMBED_DIM):
                    col = jnp.full((L,), d, jnp.int32)
                    gu = plsc.load_gather(u_rows, [row, col])
                    ge = plsc.load_gather(e_rows, [row, col])
                    acc = acc + gu * ge
                scores_v[pl.ds(h * half + j * L, L)] = acc
                return _

            lax.fori_loop(0, nblk, block, None)

        def bias_block(j, _):
            sl = pl.ds(j * L, L)
            scores_v[sl] = scores_v[sl] + ub_v[sl] + eb_v[sl]
            return _

        cub.wait()
        ceb.wait()
        lax.fori_loop(0, bpw // L, bias_block, None)
        pltpu.sync_copy(scores_v, out_hbm.at[pl.ds(base, bpw)])

    return sc_kernel


_sc_kernel = _make_sc_kernel()


def kernel(user_ids, event_ids, user_table, event_table, user_bias,
           event_bias, global_bias):
    uid = user_ids.astype(jnp.int32)
    eid = event_ids.astype(jnp.int32)
    ut2 = _tc_transpose(user_table.T)
    et2 = _tc_transpose(event_table.T)
    return _sc_kernel(uid, eid, ut2, et2,
                      user_bias.T, event_bias.T, global_bias)


# speedup vs baseline: 15.6438x; 1.2702x over previous
import functools

import jax
import jax.numpy as jnp
from jax import lax
from jax.experimental import pallas as pl
from jax.experimental.pallas import tpu as pltpu
from jax.experimental.pallas import tpu_sc as plsc

NUM_ROWS = 1000000
EMBED_DIM = 64
BATCH = 16384
PADDED = 128

L = 16
TCHUNK = 8192
H = TCHUNK // 2
NBLOCKS = (NUM_ROWS + TCHUNK - 1) // TCHUNK
PACKED_ROWS = NBLOCKS * H


def _tp_body(x_ref, o_ref):
    eye = jnp.eye(EMBED_DIM, dtype=jnp.float32)
    dn = (((0,), (0,)), ((), ()))
    o_ref[:, 0:EMBED_DIM] = jax.lax.dot_general(
        x_ref[:, 0:H], eye, dn, preferred_element_type=jnp.float32)
    o_ref[:, EMBED_DIM:PADDED] = jax.lax.dot_general(
        x_ref[:, H:TCHUNK], eye, dn, preferred_element_type=jnp.float32)


_tc_transpose = pl.pallas_call(
    _tp_body,
    grid=(NBLOCKS,),
    in_specs=[pl.BlockSpec((EMBED_DIM, TCHUNK), lambda i: (0, i))],
    out_specs=pl.BlockSpec((H, PADDED), lambda i: (i, 0)),
    out_shape=jax.ShapeDtypeStruct((PACKED_ROWS, PADDED), jnp.float32),
)


def _make_sc_kernel():
    info = plsc.get_sparse_core_info()
    nc, ns = info.num_cores, info.num_subcores
    nw = nc * ns
    bpw = BATCH // nw
    half = bpw // 2
    nblk = half // L

    mesh = plsc.VectorSubcoreMesh(core_axis_name="c", subcore_axis_name="s")

    @functools.partial(
        pl.kernel,
        mesh=mesh,
        out_type=jax.ShapeDtypeStruct((BATCH,), jnp.float32),
        scratch_types=[
            pltpu.VMEM((bpw,), jnp.int32),
            pltpu.VMEM((bpw,), jnp.int32),
            pltpu.VMEM((bpw,), jnp.int32),
            pltpu.VMEM((bpw,), jnp.int32),
            pltpu.VMEM((bpw,), jnp.int32),
            pltpu.VMEM((bpw,), jnp.int32),
            pltpu.VMEM((half, PADDED), jnp.float32),
            pltpu.VMEM((half, PADDED), jnp.float32),
            pltpu.VMEM((bpw,), jnp.float32),
            pltpu.VMEM((bpw,), jnp.float32),
            pltpu.VMEM((L,), jnp.float32),
            pltpu.VMEM((bpw,), jnp.float32),
            pltpu.SemaphoreType.DMA,
            pltpu.SemaphoreType.DMA,
            pltpu.SemaphoreType.DMA,
        ],
        compiler_params=pltpu.CompilerParams(
            needs_layout_passes=False, use_tc_tiling_on_sc=False),
    )
    def sc_kernel(uid_hbm, eid_hbm, ut_hbm, et_hbm, ub_hbm, eb_hbm, gb_hbm,
                  out_hbm, uid_v, eid_v, idxu_v, idxe_v, cbu_v, cbe_v,
                  u_rows, e_rows, ub_v, eb_v, gb_v, scores_v,
                  sem0, sem1, semb):
        wid = lax.axis_index("s") * nc + lax.axis_index("c")
        base = wid * bpw

        pltpu.sync_copy(uid_hbm.at[pl.ds(base, bpw)], uid_v)
        pltpu.sync_copy(eid_hbm.at[pl.ds(base, bpw)], eid_v)
        cub = pltpu.async_copy(ub_hbm.at[0].at[uid_v], ub_v, semb)
        ceb = pltpu.async_copy(eb_hbm.at[0].at[eid_v], eb_v, semb)
        pltpu.sync_copy(gb_hbm.at[pl.ds(0, 1)], gb_v.at[pl.ds(0, 1)])

        def remap(j, _):
            sl = pl.ds(j * L, L)
            u = uid_v[sl]
            idxu_v[sl] = ((u >> 13) << 12) + (u & (H - 1))
            cbu_v[sl] = ((u >> 12) & 1) << 6
            e = eid_v[sl]
            idxe_v[sl] = ((e >> 13) << 12) + (e & (H - 1))
            cbe_v[sl] = ((e >> 12) & 1) << 6
            return _

        lax.fori_loop(0, bpw // L, remap, None)

        gb = gb_v[pl.ds(0, L)][0]
        lane = lax.iota(jnp.int32, L)

        for h in range(2):
            cu = pltpu.async_copy(
                ut_hbm.at[idxu_v.at[pl.ds(h * half, half)]], u_rows, sem0)
            ce = pltpu.async_copy(
                et_hbm.at[idxe_v.at[pl.ds(h * half, half)]], e_rows, sem1)
            cu.wait()
            ce.wait()

            def block(j, _):
                row = jnp.full((L,), j * L, jnp.int32) + lane
                cbu = cbu_v[pl.ds(h * half + j * L, L)]
                cbe = cbe_v[pl.ds(h * half + j * L, L)]
                acc = jnp.full((L,), gb, jnp.float32)
                for d in range(EMBED_DIM):
                    gu = plsc.load_gather(u_rows, [row, cbu + d])
                    ge = plsc.load_gather(e_rows, [row, cbe + d])
                    acc = acc + gu * ge
                scores_v[pl.ds(h * half + j * L, L)] = acc
                return _

            lax.fori_loop(0, nblk, block, None)

        def bias_block(j, _):
            sl = pl.ds(j * L, L)
            scores_v[sl] = scores_v[sl] + ub_v[sl] + eb_v[sl]
            return _

        cub.wait()
        ceb.wait()
        lax.fori_loop(0, bpw // L, bias_block, None)
        pltpu.sync_copy(scores_v, out_hbm.at[pl.ds(base, bpw)])

    return sc_kernel


_sc_kernel = _make_sc_kernel()


def kernel(user_ids, event_ids, user_table, event_table, user_bias,
           event_bias, global_bias):
    uid = user_ids.astype(jnp.int32)
    eid = event_ids.astype(jnp.int32)
    ut2 = _tc_transpose(user_table.T)
    et2 = _tc_transpose(event_table.T)
    return _sc_kernel(uid, eid, ut2, et2,
                      user_bias.reshape(1, -1), event_bias.reshape(1, -1),
                      global_bias)


# speedup vs baseline: 17.3586x; 1.1096x over previous
import functools

import jax
import jax.numpy as jnp
from jax import lax
from jax.experimental import pallas as pl
from jax.experimental.pallas import tpu as pltpu
from jax.experimental.pallas import tpu_sc as plsc

NUM_ROWS = 1000000
EMBED_DIM = 64
BATCH = 16384
PADDED = 128

L = 16
TCHUNK = 16384
H = TCHUNK // 2
NBLOCKS = (NUM_ROWS + TCHUNK - 1) // TCHUNK
PACKED_ROWS = NBLOCKS * H


def _tp_body(x_ref, o_ref):
    eye = jnp.eye(EMBED_DIM, dtype=jnp.float32)
    dn = (((0,), (0,)), ((), ()))
    o_ref[:, 0:EMBED_DIM] = jax.lax.dot_general(
        x_ref[:, 0:H], eye, dn, preferred_element_type=jnp.float32)
    o_ref[:, EMBED_DIM:PADDED] = jax.lax.dot_general(
        x_ref[:, H:TCHUNK], eye, dn, preferred_element_type=jnp.float32)


_tc_transpose = pl.pallas_call(
    _tp_body,
    grid=(NBLOCKS,),
    in_specs=[pl.BlockSpec((EMBED_DIM, TCHUNK), lambda i: (0, i))],
    out_specs=pl.BlockSpec((H, PADDED), lambda i: (i, 0)),
    out_shape=jax.ShapeDtypeStruct((PACKED_ROWS, PADDED), jnp.float32),
)


def _make_sc_kernel():
    info = plsc.get_sparse_core_info()
    nc, ns = info.num_cores, info.num_subcores
    nw = nc * ns
    bpw = BATCH // nw
    half = bpw // 2
    nblk = half // L

    mesh = plsc.VectorSubcoreMesh(core_axis_name="c", subcore_axis_name="s")

    @functools.partial(
        pl.kernel,
        mesh=mesh,
        out_type=jax.ShapeDtypeStruct((BATCH,), jnp.float32),
        scratch_types=[
            pltpu.VMEM((bpw,), jnp.int32),
            pltpu.VMEM((bpw,), jnp.int32),
            pltpu.VMEM((bpw,), jnp.int32),
            pltpu.VMEM((bpw,), jnp.int32),
            pltpu.VMEM((bpw,), jnp.int32),
            pltpu.VMEM((bpw,), jnp.int32),
            pltpu.VMEM((half, PADDED), jnp.float32),
            pltpu.VMEM((half, PADDED), jnp.float32),
            pltpu.VMEM((bpw,), jnp.float32),
            pltpu.VMEM((bpw,), jnp.float32),
            pltpu.VMEM((L,), jnp.float32),
            pltpu.VMEM((bpw,), jnp.float32),
            pltpu.SemaphoreType.DMA,
            pltpu.SemaphoreType.DMA,
            pltpu.SemaphoreType.DMA,
        ],
        compiler_params=pltpu.CompilerParams(
            needs_layout_passes=False, use_tc_tiling_on_sc=False),
    )
    def sc_kernel(uid_hbm, eid_hbm, ut_hbm, et_hbm, ub_hbm, eb_hbm, gb_hbm,
                  out_hbm, uid_v, eid_v, idxu_v, idxe_v, cbu_v, cbe_v,
                  u_rows, e_rows, ub_v, eb_v, gb_v, scores_v,
                  sem0, sem1, semb):
        wid = lax.axis_index("s") * nc + lax.axis_index("c")
        base = wid * bpw

        pltpu.sync_copy(uid_hbm.at[pl.ds(base, bpw)], uid_v)
        pltpu.sync_copy(eid_hbm.at[pl.ds(base, bpw)], eid_v)
        cub = pltpu.async_copy(ub_hbm.at[0].at[uid_v], ub_v, semb)
        ceb = pltpu.async_copy(eb_hbm.at[0].at[eid_v], eb_v, semb)
        pltpu.sync_copy(gb_hbm.at[pl.ds(0, 1)], gb_v.at[pl.ds(0, 1)])

        def remap(j, _):
            sl = pl.ds(j * L, L)
            u = uid_v[sl]
            idxu_v[sl] = ((u >> 14) << 13) + (u & (H - 1))
            cbu_v[sl] = ((u >> 13) & 1) << 6
            e = eid_v[sl]
            idxe_v[sl] = ((e >> 14) << 13) + (e & (H - 1))
            cbe_v[sl] = ((e >> 13) & 1) << 6
            return _

        lax.fori_loop(0, bpw // L, remap, None)

        gb = gb_v[pl.ds(0, L)][0]
        lane = lax.iota(jnp.int32, L)

        for h in range(2):
            cu = pltpu.async_copy(
                ut_hbm.at[idxu_v.at[pl.ds(h * half, half)]], u_rows, sem0)
            ce = pltpu.async_copy(
                et_hbm.at[idxe_v.at[pl.ds(h * half, half)]], e_rows, sem1)
            cu.wait()
            ce.wait()

            def block(j, _):
                row = jnp.full((L,), j * L, jnp.int32) + lane
                cbu = cbu_v[pl.ds(h * half + j * L, L)]
                cbe = cbe_v[pl.ds(h * half + j * L, L)]
                acc = jnp.full((L,), gb, jnp.float32)
                for d in range(EMBED_DIM):
                    gu = plsc.load_gather(u_rows, [row, cbu + d])
                    ge = plsc.load_gather(e_rows, [row, cbe + d])
                    acc = acc + gu * ge
                scores_v[pl.ds(h * half + j * L, L)] = acc
                return _

            lax.fori_loop(0, nblk, block, None)

        def bias_block(j, _):
            sl = pl.ds(j * L, L)
            scores_v[sl] = scores_v[sl] + ub_v[sl] + eb_v[sl]
            return _

        cub.wait()
        ceb.wait()
        lax.fori_loop(0, bpw // L, bias_block, None)
        pltpu.sync_copy(scores_v, out_hbm.at[pl.ds(base, bpw)])

    return sc_kernel


_sc_kernel = _make_sc_kernel()


def kernel(user_ids, event_ids, user_table, event_table, user_bias,
           event_bias, global_bias):
    uid = user_ids.astype(jnp.int32)
    eid = event_ids.astype(jnp.int32)
    ut2 = _tc_transpose(user_table.T)
    et2 = _tc_transpose(event_table.T)
    return _sc_kernel(uid, eid, ut2, et2,
                      user_bias.reshape(1, -1), event_bias.reshape(1, -1),
                      global_bias)
